# Initial kernel scaffold; baseline (speedup 1.0000x reference)
import jax
import jax.numpy as jnp
from jax.experimental import pallas as pl


def kernel(h_in, edge_index, edge_attr, H1, H2, Wr0, Wr1, Wr2, Wout, bout):
    raise NotImplementedError("write your pallas kernel here")


# speedup vs baseline: 1.0499x; 1.0499x over previous
import jax
import jax.numpy as jnp
from jax import lax
from jax.experimental import pallas as pl
from jax.experimental.pallas import tpu as pltpu

_N, _E, _D, _DE = 10000, 320000, 128, 4
_OUT0, _OUT1, _OUT2, _LT, _NDEG = 64, 64, 128, 12, 64

_BN = 2000


def _deg_update(aggh, agga, di, ha_ref, hb_ref, out_w):
    def dstep(d, acc):
        m = di == d
        am = jnp.where(m, aggh, 0.0)
        aa = jnp.where(m, agga, 0.0)
        return (acc
                + jnp.dot(am, ha_ref[d], preferred_element_type=jnp.float32)
                + jnp.dot(aa, hb_ref[d], preferred_element_type=jnp.float32))

    acc = lax.fori_loop(0, _NDEG, dstep,
                        jnp.zeros((aggh.shape[0], out_w), jnp.float32))
    return jax.nn.sigmoid(acc)


def _tc1_body(ph_ref, pa_ref, deg_ref, ha_ref, hb_ref, h1_ref):
    di = jnp.minimum(deg_ref[...], _NDEG - 1)
    h1_ref[...] = _deg_update(ph_ref[...], pa_ref[...], di,
                              ha_ref, hb_ref, _OUT0)


def _tc1(agg_h, agg_a, deg, h1a, h1b):
    grid = (_N // _BN,)
    return pl.pallas_call(
        _tc1_body,
        grid=grid,
        in_specs=[
            pl.BlockSpec((_BN, _D), lambda i: (i, 0)),
            pl.BlockSpec((_BN, _DE), lambda i: (i, 0)),
            pl.BlockSpec((_BN, 1), lambda i: (i, 0)),
            pl.BlockSpec(h1a.shape, lambda i: (0, 0, 0)),
            pl.BlockSpec(h1b.shape, lambda i: (0, 0, 0)),
        ],
        out_specs=pl.BlockSpec((_BN, _OUT0), lambda i: (i, 0)),
        out_shape=jax.ShapeDtypeStruct((_N, _OUT0), jnp.float32),
    )(agg_h, agg_a, deg, h1a, h1b)


def _softmax_colsum(x, w_ref):
    y = jnp.dot(x, w_ref[...], preferred_element_type=jnp.float32)
    y = y - jnp.max(y, axis=1, keepdims=True)
    e = jnp.exp(y)
    p = e / jnp.sum(e, axis=1, keepdims=True)
    return jnp.sum(p, axis=0, keepdims=True)


def _tc2_body(p2_ref, pa_ref, deg_ref, hin_ref, h1_ref, h2a_ref, h2b_ref,
              wr0_ref, wr1_ref, wr2_ref, wout_ref, bout_ref,
              out_ref, racc_ref):
    i = pl.program_id(0)
    di = jnp.minimum(deg_ref[...], _NDEG - 1)
    h2 = _deg_update(p2_ref[...], pa_ref[...], di, h2a_ref, h2b_ref, _OUT1)
    r = (_softmax_colsum(hin_ref[...], wr0_ref)
         + _softmax_colsum(h1_ref[...], wr1_ref)
         + _softmax_colsum(h2, wr2_ref))

    @pl.when(i == 0)
    def _():
        racc_ref[...] = r

    @pl.when(i > 0)
    def _():
        racc_ref[...] = racc_ref[...] + r

    @pl.when(i == pl.num_programs(0) - 1)
    def _():
        out_ref[...] = (jnp.dot(racc_ref[...], wout_ref[...],
                                preferred_element_type=jnp.float32)
                        + bout_ref[...])


def _tc2(agg2, agg_a, deg, h_in, h1, h2a, h2b, Wr0, Wr1, Wr2, Wout, bout2):
    grid = (_N // _BN,)
    return pl.pallas_call(
        _tc2_body,
        grid=grid,
        in_specs=[
            pl.BlockSpec((_BN, _OUT0), lambda i: (i, 0)),
            pl.BlockSpec((_BN, _DE), lambda i: (i, 0)),
            pl.BlockSpec((_BN, 1), lambda i: (i, 0)),
            pl.BlockSpec((_BN, _D), lambda i: (i, 0)),
            pl.BlockSpec((_BN, _OUT0), lambda i: (i, 0)),
            pl.BlockSpec(h2a.shape, lambda i: (0, 0, 0)),
            pl.BlockSpec(h2b.shape, lambda i: (0, 0, 0)),
            pl.BlockSpec(Wr0.shape, lambda i: (0, 0)),
            pl.BlockSpec(Wr1.shape, lambda i: (0, 0)),
            pl.BlockSpec(Wr2.shape, lambda i: (0, 0)),
            pl.BlockSpec(Wout.shape, lambda i: (0, 0)),
            pl.BlockSpec((1, _LT), lambda i: (0, 0)),
        ],
        out_specs=pl.BlockSpec((1, _LT), lambda i: (0, 0)),
        out_shape=jax.ShapeDtypeStruct((1, _LT), jnp.float32),
        scratch_shapes=[pltpu.VMEM((1, _OUT2), jnp.float32)],
    )(agg2, agg_a, deg, h_in, h1, h2a, h2b, Wr0, Wr1, Wr2, Wout, bout2)


def kernel(h_in, edge_index, edge_attr, H1, H2, Wr0, Wr1, Wr2, Wout, bout):
    src = edge_index[0]
    dst = edge_index[1]

    m1 = jnp.concatenate(
        [jnp.take(h_in, src, axis=0), edge_attr,
         jnp.ones((_E, 1), jnp.float32)], axis=1)
    agg1 = jax.ops.segment_sum(m1, dst, num_segments=_N)
    agg_h = agg1[:, :_D]
    agg_a = agg1[:, _D:_D + _DE]
    deg = agg1[:, _D + _DE:].astype(jnp.int32)

    h1a, h1b = H1[:, :_D, :], H1[:, _D:, :]
    h1 = _tc1(agg_h, agg_a, deg, h1a, h1b)

    agg2 = jax.ops.segment_sum(jnp.take(h1, src, axis=0), dst,
                               num_segments=_N)

    h2a, h2b = H2[:, :_OUT0, :], H2[:, _OUT0:, :]
    out = _tc2(agg2, agg_a, deg, h_in, h1, h2a, h2b,
               Wr0, Wr1, Wr2, Wout, bout.reshape(1, _LT))
    return out.reshape(_LT)
